# baseline (speedup 1.0000x reference)
import functools

import jax
import jax.numpy as jnp
from jax import lax
from jax.experimental import pallas as pl
from jax.experimental.pallas import tpu as pltpu
from jax.experimental.pallas import tpu_sc as plsc

N = 10000
E = 320000
D = 128

NC = 2
NS = 16
NW = NC * NS
EPW = E // NW
C = 80
NCHUNK = EPW // C
N2 = 10240
RPT = N2 // NS
ZR = 64

_MESH = plsc.VectorSubcoreMesh(
    core_axis_name="c", subcore_axis_name="s", num_cores=NC, num_subcores=NS
)


def _sc_agg_body(with_deg, h_hbm, src_hbm, dst_hbm, zrows_hbm, *refs):
    if with_deg:
        (zdeg_hbm, ones_hbm, agg_out, deg_out,
         sidx, didx, rows, agg_sh, sem, ones_v, deg_sh) = refs
    else:
        (agg_out, sidx, didx, rows, agg_sh, sem) = refs
    cid = lax.axis_index("c")
    sid = lax.axis_index("s")
    wid = cid * NS + sid

    pltpu.sync_copy(zrows_hbm, rows.at[pl.ds(0, ZR)])
    zbase = sid * RPT
    for j in range(RPT // ZR):
        pltpu.sync_copy(
            rows.at[pl.ds(0, ZR)], agg_sh.at[pl.ds(zbase + j * ZR, ZR)]
        )
    if with_deg:
        pltpu.sync_copy(ones_hbm, ones_v)
        pltpu.sync_copy(zdeg_hbm, deg_sh.at[pl.ds(zbase, RPT)])
    plsc.subcore_barrier()

    pltpu.sync_copy(src_hbm.at[wid], sidx)
    pltpu.sync_copy(dst_hbm.at[wid], didx)

    def eloop(i, carry):
        pltpu.async_copy(h_hbm.at[sidx.at[i]], rows, sem).wait()
        pltpu.sync_copy(rows, agg_sh.at[didx.at[i]], add=True)
        if with_deg:
            pltpu.sync_copy(ones_v, deg_sh.at[didx.at[i]], add=True)
        return carry

    lax.fori_loop(0, NCHUNK, eloop, 0)
    plsc.subcore_barrier()

    pltpu.sync_copy(
        agg_sh.at[pl.ds(zbase, RPT)], agg_out.at[cid, pl.ds(zbase, RPT)]
    )
    if with_deg:
        pltpu.sync_copy(
            deg_sh.at[pl.ds(zbase, RPT)], deg_out.at[cid, pl.ds(zbase, RPT)]
        )


def _make_sc_agg(with_deg):
    agg_t = jax.ShapeDtypeStruct((NC, N2, D), jnp.float32)
    out_type = [agg_t] if with_deg else agg_t
    scratch = [
        pltpu.VMEM((NCHUNK, C), jnp.int32),
        pltpu.VMEM((NCHUNK, C), jnp.int32),
        pltpu.VMEM((C, D), jnp.float32),
        pltpu.VMEM_SHARED((N2, D), jnp.float32),
        pltpu.SemaphoreType.DMA,
    ]
    if with_deg:
        out_type.append(jax.ShapeDtypeStruct((NC, N2, 16), jnp.float32))
        scratch.append(pltpu.VMEM((C, 16), jnp.float32))
        scratch.append(pltpu.VMEM_SHARED((N2, 16), jnp.float32))
    return pl.kernel(
        functools.partial(_sc_agg_body, with_deg),
        out_type=out_type,
        mesh=_MESH,
        scratch_types=scratch,
        compiler_params=pltpu.CompilerParams(use_tc_tiling_on_sc=False),
    )


_sc_agg_deg = _make_sc_agg(True)
_sc_agg = _make_sc_agg(False)


def _dense_body(relu_norm, h_ref, agg_ref, degt_ref, ws_ref, wn_ref, b_ref,
                o_ref):
    h = h_ref[...]
    agg = agg_ref[0] + agg_ref[1]
    deg = degt_ref[0][:, 0:1] + degt_ref[1][:, 0:1]
    deg = jnp.maximum(deg, 1.0)
    hn = agg / deg
    out = jnp.dot(h, ws_ref[...], preferred_element_type=jnp.float32)
    out = out + jnp.dot(hn, wn_ref[...], preferred_element_type=jnp.float32)
    out = out + b_ref[...]
    if relu_norm:
        out = jnp.maximum(out, 0.0)
        nrm = jnp.sqrt(jnp.sum(out * out, axis=-1, keepdims=True))
        out = out / jnp.maximum(nrm, 1e-12)
    o_ref[...] = out


R = 1000


def _dense(h, aggp, degt, Ws, Wn, b, relu_norm):
    return pl.pallas_call(
        functools.partial(_dense_body, relu_norm),
        grid=(N // R,),
        in_specs=[
            pl.BlockSpec((R, D), lambda i: (i, 0)),
            pl.BlockSpec((NC, R, D), lambda i: (0, i, 0)),
            pl.BlockSpec((NC, R, 16), lambda i: (0, i, 0)),
            pl.BlockSpec((D, D), lambda i: (0, 0)),
            pl.BlockSpec((D, D), lambda i: (0, 0)),
            pl.BlockSpec((1, D), lambda i: (0, 0)),
        ],
        out_specs=pl.BlockSpec((R, D), lambda i: (i, 0)),
        out_shape=jax.ShapeDtypeStruct((N, D), jnp.float32),
    )(h, aggp, degt, Ws, Wn, b.reshape(1, D))


def kernel(x, edge_index, W_self0, W_neigh0, b0, W_self1, W_neigh1, b1,
           W_self2, W_neigh2, b2):
    src = edge_index[0].reshape(NW, NCHUNK, C)
    dst = edge_index[1].reshape(NW, NCHUNK, C)
    zrows = jnp.zeros((ZR, D), jnp.float32)

    zdeg = jnp.zeros((RPT, 16), jnp.float32)
    ones = jnp.ones((C, 16), jnp.float32)

    aggp, degt = _sc_agg_deg(x, src, dst, zrows, zdeg, ones)
    h = _dense(x, aggp, degt, W_self0, W_neigh0, b0, True)
    aggp = _sc_agg(h, src, dst, zrows)
    h = _dense(h, aggp, degt, W_self1, W_neigh1, b1, True)
    aggp = _sc_agg(h, src, dst, zrows)
    return _dense(h, aggp, degt, W_self2, W_neigh2, b2, False)


# speedup vs baseline: 1.3241x; 1.3241x over previous
import functools

import jax
import jax.numpy as jnp
from jax import lax
from jax.experimental import pallas as pl
from jax.experimental.pallas import tpu as pltpu
from jax.experimental.pallas import tpu_sc as plsc

N = 10000
E = 320000
D = 128

NC = 2
NS = 16
NW = NC * NS
EPW = E // NW
C = 80
NCHUNK = EPW // C
N2 = 10240
RPT = N2 // NS
ZR = 64

_MESH = plsc.VectorSubcoreMesh(
    core_axis_name="c", subcore_axis_name="s", num_cores=NC, num_subcores=NS
)


def _sc_agg_body(with_deg, h_hbm, src_hbm, dst_hbm, zrows_hbm, *refs):
    if with_deg:
        (zdeg_hbm, ones_hbm, agg_out, deg_out,
         sidx, didx, rows, agg_sh, sem, ones_v, deg_sh) = refs
        rows1 = sem1 = None
    else:
        (agg_out, sidx, didx, rows, agg_sh, sem, rows1, sem1) = refs
    cid = lax.axis_index("c")
    sid = lax.axis_index("s")
    wid = cid * NS + sid

    pltpu.sync_copy(zrows_hbm, rows.at[pl.ds(0, ZR)])
    zbase = sid * RPT
    for j in range(RPT // ZR):
        pltpu.sync_copy(
            rows.at[pl.ds(0, ZR)], agg_sh.at[pl.ds(zbase + j * ZR, ZR)]
        )
    if with_deg:
        pltpu.sync_copy(ones_hbm, ones_v)
        pltpu.sync_copy(zdeg_hbm, deg_sh.at[pl.ds(zbase, RPT)])
    plsc.subcore_barrier()

    pltpu.sync_copy(src_hbm.at[wid], sidx)
    pltpu.sync_copy(dst_hbm.at[wid], didx)

    if with_deg:
        def eloop(i, carry):
            pltpu.async_copy(h_hbm.at[sidx.at[i]], rows, sem).wait()
            pltpu.sync_copy(rows, agg_sh.at[didx.at[i]], add=True)
            pltpu.sync_copy(ones_v, deg_sh.at[didx.at[i]], add=True)
            return carry

        lax.fori_loop(0, NCHUNK, eloop, 0)
    else:
        def gstart(j, buf, gsem):
            pltpu.async_copy(h_hbm.at[sidx.at[j]], buf, gsem)

        def gwait(buf, gsem):
            pltpu.make_async_copy(h_hbm.at[sidx.at[0]], buf, gsem).wait()

        gstart(0, rows, sem)

        def eloop(g, carry):
            j = 2 * g
            gstart(j + 1, rows1, sem1)
            gwait(rows, sem)
            pltpu.sync_copy(rows, agg_sh.at[didx.at[j]], add=True)
            gstart(j + 2, rows, sem)
            gwait(rows1, sem1)
            pltpu.sync_copy(rows1, agg_sh.at[didx.at[j + 1]], add=True)
            return carry

        lax.fori_loop(0, NCHUNK // 2, eloop, 0)
        gwait(rows, sem)
        pltpu.sync_copy(rows, agg_sh.at[didx.at[NCHUNK - 1]], add=True)
    plsc.subcore_barrier()

    pltpu.sync_copy(
        agg_sh.at[pl.ds(zbase, RPT)], agg_out.at[cid, pl.ds(zbase, RPT)]
    )
    if with_deg:
        pltpu.sync_copy(
            deg_sh.at[pl.ds(zbase, RPT)], deg_out.at[cid, pl.ds(zbase, RPT)]
        )


def _make_sc_agg(with_deg):
    agg_t = jax.ShapeDtypeStruct((NC, N2, D), jnp.float32)
    out_type = [agg_t] if with_deg else agg_t
    scratch = [
        pltpu.VMEM((NCHUNK, C), jnp.int32),
        pltpu.VMEM((NCHUNK, C), jnp.int32),
        pltpu.VMEM((C, D), jnp.float32),
        pltpu.VMEM_SHARED((N2, D), jnp.float32),
        pltpu.SemaphoreType.DMA,
    ]
    if not with_deg:
        scratch.append(pltpu.VMEM((C, D), jnp.float32))
        scratch.append(pltpu.SemaphoreType.DMA)
    if with_deg:
        out_type.append(jax.ShapeDtypeStruct((NC, N2, 16), jnp.float32))
        scratch.append(pltpu.VMEM((C, 16), jnp.float32))
        scratch.append(pltpu.VMEM_SHARED((N2, 16), jnp.float32))
    return pl.kernel(
        functools.partial(_sc_agg_body, with_deg),
        out_type=out_type,
        mesh=_MESH,
        scratch_types=scratch,
        compiler_params=pltpu.CompilerParams(use_tc_tiling_on_sc=False),
    )


_sc_agg_deg = _make_sc_agg(True)
_sc_agg = _make_sc_agg(False)


def _dense_body(relu_norm, h_ref, agg_ref, degt_ref, ws_ref, wn_ref, b_ref,
                o_ref):
    h = h_ref[...]
    agg = agg_ref[0] + agg_ref[1]
    deg = degt_ref[0][:, 0:1] + degt_ref[1][:, 0:1]
    deg = jnp.maximum(deg, 1.0)
    hn = agg / deg
    out = jnp.dot(h, ws_ref[...], preferred_element_type=jnp.float32)
    out = out + jnp.dot(hn, wn_ref[...], preferred_element_type=jnp.float32)
    out = out + b_ref[...]
    if relu_norm:
        out = jnp.maximum(out, 0.0)
        nrm = jnp.sqrt(jnp.sum(out * out, axis=-1, keepdims=True))
        out = out / jnp.maximum(nrm, 1e-12)
    o_ref[...] = out


R = 1000


def _dense(h, aggp, degt, Ws, Wn, b, relu_norm):
    return pl.pallas_call(
        functools.partial(_dense_body, relu_norm),
        grid=(N // R,),
        in_specs=[
            pl.BlockSpec((R, D), lambda i: (i, 0)),
            pl.BlockSpec((NC, R, D), lambda i: (0, i, 0)),
            pl.BlockSpec((NC, R, 16), lambda i: (0, i, 0)),
            pl.BlockSpec((D, D), lambda i: (0, 0)),
            pl.BlockSpec((D, D), lambda i: (0, 0)),
            pl.BlockSpec((1, D), lambda i: (0, 0)),
        ],
        out_specs=pl.BlockSpec((R, D), lambda i: (i, 0)),
        out_shape=jax.ShapeDtypeStruct((N, D), jnp.float32),
    )(h, aggp, degt, Ws, Wn, b.reshape(1, D))


def kernel(x, edge_index, W_self0, W_neigh0, b0, W_self1, W_neigh1, b1,
           W_self2, W_neigh2, b2):
    src = edge_index[0].reshape(NW, NCHUNK, C)
    dst = edge_index[1].reshape(NW, NCHUNK, C)
    zrows = jnp.zeros((ZR, D), jnp.float32)

    zdeg = jnp.zeros((RPT, 16), jnp.float32)
    ones = jnp.ones((C, 16), jnp.float32)

    aggp, degt = _sc_agg_deg(x, src, dst, zrows, zdeg, ones)
    h = _dense(x, aggp, degt, W_self0, W_neigh0, b0, True)
    aggp = _sc_agg(h, src, dst, zrows)
    h = _dense(h, aggp, degt, W_self1, W_neigh1, b1, True)
    aggp = _sc_agg(h, src, dst, zrows)
    return _dense(h, aggp, degt, W_self2, W_neigh2, b2, False)


# speedup vs baseline: 1.5844x; 1.1965x over previous
import functools

import jax
import jax.numpy as jnp
from jax import lax
from jax.experimental import pallas as pl
from jax.experimental.pallas import tpu as pltpu
from jax.experimental.pallas import tpu_sc as plsc

N = 10000
E = 320000
D = 128

NC = 2
NS = 16
NW = NC * NS
EPW = E // NW
C = 80
NCHUNK = EPW // C
N2 = 10240
RPT = N2 // NS
ZR = 64

_MESH = plsc.VectorSubcoreMesh(
    core_axis_name="c", subcore_axis_name="s", num_cores=NC, num_subcores=NS
)


def _sc_agg_body(h_hbm, src_hbm, dst_hbm, zrows_hbm, agg_out,
                 sidx, didx, rows, agg_sh, sem, rows1, sem1):
    cid = lax.axis_index("c")
    sid = lax.axis_index("s")
    wid = cid * NS + sid

    pltpu.sync_copy(zrows_hbm, rows.at[pl.ds(0, ZR)])
    zbase = sid * RPT
    for j in range(RPT // ZR):
        pltpu.sync_copy(
            rows.at[pl.ds(0, ZR)], agg_sh.at[pl.ds(zbase + j * ZR, ZR)]
        )
    plsc.subcore_barrier()

    pltpu.sync_copy(src_hbm.at[wid], sidx)
    pltpu.sync_copy(dst_hbm.at[wid], didx)

    def gstart(j, buf, gsem):
        pltpu.async_copy(h_hbm.at[sidx.at[j]], buf, gsem)

    def gwait(buf, gsem):
        pltpu.make_async_copy(h_hbm.at[sidx.at[0]], buf, gsem).wait()

    gstart(0, rows, sem)

    def eloop(g, carry):
        j = 2 * g
        gstart(j + 1, rows1, sem1)
        gwait(rows, sem)
        pltpu.sync_copy(rows, agg_sh.at[didx.at[j]], add=True)
        gstart(j + 2, rows, sem)
        gwait(rows1, sem1)
        pltpu.sync_copy(rows1, agg_sh.at[didx.at[j + 1]], add=True)
        return carry

    lax.fori_loop(0, NCHUNK // 2, eloop, 0)
    gwait(rows, sem)
    pltpu.sync_copy(rows, agg_sh.at[didx.at[NCHUNK - 1]], add=True)
    plsc.subcore_barrier()

    pltpu.sync_copy(
        agg_sh.at[pl.ds(zbase, RPT)], agg_out.at[cid, pl.ds(zbase, RPT)]
    )


def _sc_deg_body(dst_hbm, zdeg_hbm, ones_hbm, deg_out,
                 didx, ones_v, deg_sh):
    cid = lax.axis_index("c")
    sid = lax.axis_index("s")
    wid = cid * NS + sid

    zbase = sid * RPT
    pltpu.sync_copy(ones_hbm, ones_v)
    pltpu.sync_copy(zdeg_hbm, deg_sh.at[pl.ds(zbase, RPT)])
    plsc.subcore_barrier()

    pltpu.sync_copy(dst_hbm.at[wid], didx)

    def dloop(i, carry):
        pltpu.sync_copy(ones_v, deg_sh.at[didx.at[i]], add=True)
        return carry

    lax.fori_loop(0, NCHUNK, dloop, 0)
    plsc.subcore_barrier()

    pltpu.sync_copy(
        deg_sh.at[pl.ds(zbase, RPT)], deg_out.at[cid, pl.ds(zbase, RPT)]
    )


_sc_agg = pl.kernel(
    _sc_agg_body,
    out_type=jax.ShapeDtypeStruct((NC, N2, D), jnp.float32),
    mesh=_MESH,
    scratch_types=[
        pltpu.VMEM((NCHUNK, C), jnp.int32),
        pltpu.VMEM((NCHUNK, C), jnp.int32),
        pltpu.VMEM((C, D), jnp.float32),
        pltpu.VMEM_SHARED((N2, D), jnp.float32),
        pltpu.SemaphoreType.DMA,
        pltpu.VMEM((C, D), jnp.float32),
        pltpu.SemaphoreType.DMA,
    ],
    compiler_params=pltpu.CompilerParams(use_tc_tiling_on_sc=False),
)

_sc_deg = pl.kernel(
    _sc_deg_body,
    out_type=jax.ShapeDtypeStruct((NC, N2, 16), jnp.float32),
    mesh=_MESH,
    scratch_types=[
        pltpu.VMEM((NCHUNK, C), jnp.int32),
        pltpu.VMEM((C, 16), jnp.float32),
        pltpu.VMEM_SHARED((N2, 16), jnp.float32),
    ],
    compiler_params=pltpu.CompilerParams(use_tc_tiling_on_sc=False),
)


def _dense_body(relu_norm, h_ref, agg_ref, degt_ref, ws_ref, wn_ref, b_ref,
                o_ref):
    h = h_ref[...]
    agg = agg_ref[0] + agg_ref[1]
    deg = degt_ref[0][:, 0:1] + degt_ref[1][:, 0:1]
    deg = jnp.maximum(deg, 1.0)
    hn = agg / deg
    out = jnp.dot(h, ws_ref[...], preferred_element_type=jnp.float32)
    out = out + jnp.dot(hn, wn_ref[...], preferred_element_type=jnp.float32)
    out = out + b_ref[...]
    if relu_norm:
        out = jnp.maximum(out, 0.0)
        nrm = jnp.sqrt(jnp.sum(out * out, axis=-1, keepdims=True))
        out = out / jnp.maximum(nrm, 1e-12)
    o_ref[...] = out


R = 1000


def _dense(h, aggp, degt, Ws, Wn, b, relu_norm):
    return pl.pallas_call(
        functools.partial(_dense_body, relu_norm),
        grid=(N // R,),
        in_specs=[
            pl.BlockSpec((R, D), lambda i: (i, 0)),
            pl.BlockSpec((NC, R, D), lambda i: (0, i, 0)),
            pl.BlockSpec((NC, R, 16), lambda i: (0, i, 0)),
            pl.BlockSpec((D, D), lambda i: (0, 0)),
            pl.BlockSpec((D, D), lambda i: (0, 0)),
            pl.BlockSpec((1, D), lambda i: (0, 0)),
        ],
        out_specs=pl.BlockSpec((R, D), lambda i: (i, 0)),
        out_shape=jax.ShapeDtypeStruct((N, D), jnp.float32),
    )(h, aggp, degt, Ws, Wn, b.reshape(1, D))


def kernel(x, edge_index, W_self0, W_neigh0, b0, W_self1, W_neigh1, b1,
           W_self2, W_neigh2, b2):
    src = edge_index[0].reshape(NW, NCHUNK, C)
    dst = edge_index[1].reshape(NW, NCHUNK, C)
    zrows = jnp.zeros((ZR, D), jnp.float32)

    zdeg = jnp.zeros((RPT, 16), jnp.float32)
    ones = jnp.ones((C, 16), jnp.float32)

    degt = _sc_deg(dst, zdeg, ones)
    aggp = _sc_agg(x, src, dst, zrows)
    h = _dense(x, aggp, degt, W_self0, W_neigh0, b0, True)
    aggp = _sc_agg(h, src, dst, zrows)
    h = _dense(h, aggp, degt, W_self1, W_neigh1, b1, True)
    aggp = _sc_agg(h, src, dst, zrows)
    return _dense(h, aggp, degt, W_self2, W_neigh2, b2, False)


# speedup vs baseline: 1.5920x; 1.0048x over previous
import functools

import jax
import jax.numpy as jnp
from jax import lax
from jax.experimental import pallas as pl
from jax.experimental.pallas import tpu as pltpu
from jax.experimental.pallas import tpu_sc as plsc

N = 10000
E = 320000
D = 128

NC = 2
NS = 16
NW = NC * NS
EPW = E // NW
C = 80
NCHUNK = EPW // C
N2 = 10240
RPT = N2 // NS
ZR = 64

_MESH = plsc.VectorSubcoreMesh(
    core_axis_name="c", subcore_axis_name="s", num_cores=NC, num_subcores=NS
)


def _sc_agg_body(h_hbm, src_hbm, dst_hbm, zrows_hbm, agg_out,
                 sidx, didx, rows, agg_sh, sem, rows1, sem1, zbuf):
    cid = lax.axis_index("c")
    sid = lax.axis_index("s")
    wid = cid * NS + sid

    pltpu.sync_copy(src_hbm.at[wid], sidx)
    pltpu.sync_copy(dst_hbm.at[wid], didx)

    H = C // 2

    def gstart(j, buf, gsem):
        pltpu.async_copy(
            h_hbm.at[sidx.at[j, pl.ds(0, H)]], buf.at[pl.ds(0, H)], gsem)
        pltpu.async_copy(
            h_hbm.at[sidx.at[j, pl.ds(H, H)]], buf.at[pl.ds(H, H)], gsem)

    def gwait(buf, gsem):
        pltpu.make_async_copy(h_hbm.at[sidx.at[0]], buf, gsem).wait()

    gstart(0, rows, sem)
    gstart(1, rows1, sem1)

    pltpu.sync_copy(zrows_hbm, zbuf)
    zbase = sid * RPT
    for j in range(RPT // ZR):
        pltpu.sync_copy(zbuf, agg_sh.at[pl.ds(zbase + j * ZR, ZR)])
    plsc.subcore_barrier()

    def eloop(g, carry):
        j = 2 * g
        gwait(rows, sem)
        pltpu.sync_copy(rows, agg_sh.at[didx.at[j]], add=True)
        gstart(j + 2, rows, sem)
        gwait(rows1, sem1)
        pltpu.sync_copy(rows1, agg_sh.at[didx.at[j + 1]], add=True)
        gstart(j + 3, rows1, sem1)
        return carry

    lax.fori_loop(0, NCHUNK // 2 - 1, eloop, 0)
    gwait(rows, sem)
    pltpu.sync_copy(rows, agg_sh.at[didx.at[NCHUNK - 3]], add=True)
    gstart(NCHUNK - 1, rows, sem)
    gwait(rows1, sem1)
    pltpu.sync_copy(rows1, agg_sh.at[didx.at[NCHUNK - 2]], add=True)
    gwait(rows, sem)
    pltpu.sync_copy(rows, agg_sh.at[didx.at[NCHUNK - 1]], add=True)
    plsc.subcore_barrier()

    pltpu.sync_copy(
        agg_sh.at[pl.ds(zbase, RPT)], agg_out.at[cid, pl.ds(zbase, RPT)]
    )


def _sc_deg_body(dst_hbm, zdeg_hbm, ones_hbm, deg_out,
                 didx, ones_v, deg_sh):
    cid = lax.axis_index("c")
    sid = lax.axis_index("s")
    wid = cid * NS + sid

    zbase = sid * RPT
    pltpu.sync_copy(ones_hbm, ones_v)
    pltpu.sync_copy(zdeg_hbm, deg_sh.at[pl.ds(zbase, RPT)])
    plsc.subcore_barrier()

    pltpu.sync_copy(dst_hbm.at[wid], didx)

    def dloop(i, carry):
        pltpu.sync_copy(ones_v, deg_sh.at[didx.at[i]], add=True)
        return carry

    lax.fori_loop(0, NCHUNK, dloop, 0)
    plsc.subcore_barrier()

    pltpu.sync_copy(
        deg_sh.at[pl.ds(zbase, RPT)], deg_out.at[cid, pl.ds(zbase, RPT)]
    )


_sc_agg = pl.kernel(
    _sc_agg_body,
    out_type=jax.ShapeDtypeStruct((NC, N2, D), jnp.float32),
    mesh=_MESH,
    scratch_types=[
        pltpu.VMEM((NCHUNK, C), jnp.int32),
        pltpu.VMEM((NCHUNK, C), jnp.int32),
        pltpu.VMEM((C, D), jnp.float32),
        pltpu.VMEM_SHARED((N2, D), jnp.float32),
        pltpu.SemaphoreType.DMA,
        pltpu.VMEM((C, D), jnp.float32),
        pltpu.SemaphoreType.DMA,
        pltpu.VMEM((ZR, D), jnp.float32),
    ],
    compiler_params=pltpu.CompilerParams(use_tc_tiling_on_sc=False),
)

_sc_deg = pl.kernel(
    _sc_deg_body,
    out_type=jax.ShapeDtypeStruct((NC, N2, 16), jnp.float32),
    mesh=_MESH,
    scratch_types=[
        pltpu.VMEM((NCHUNK, C), jnp.int32),
        pltpu.VMEM((C, 16), jnp.float32),
        pltpu.VMEM_SHARED((N2, 16), jnp.float32),
    ],
    compiler_params=pltpu.CompilerParams(use_tc_tiling_on_sc=False),
)


def _dense_body(relu_norm, h_ref, agg_ref, degt_ref, ws_ref, wn_ref, b_ref,
                o_ref):
    h = h_ref[...]
    agg = agg_ref[0] + agg_ref[1]
    deg = degt_ref[0][:, 0:1] + degt_ref[1][:, 0:1]
    deg = jnp.maximum(deg, 1.0)
    hn = agg / deg
    out = jnp.dot(h, ws_ref[...], preferred_element_type=jnp.float32)
    out = out + jnp.dot(hn, wn_ref[...], preferred_element_type=jnp.float32)
    out = out + b_ref[...]
    if relu_norm:
        out = jnp.maximum(out, 0.0)
        nrm = jnp.sqrt(jnp.sum(out * out, axis=-1, keepdims=True))
        out = out / jnp.maximum(nrm, 1e-12)
    o_ref[...] = out


R = 1000


def _dense(h, aggp, degt, Ws, Wn, b, relu_norm):
    return pl.pallas_call(
        functools.partial(_dense_body, relu_norm),
        grid=(N // R,),
        in_specs=[
            pl.BlockSpec((R, D), lambda i: (i, 0)),
            pl.BlockSpec((NC, R, D), lambda i: (0, i, 0)),
            pl.BlockSpec((NC, R, 16), lambda i: (0, i, 0)),
            pl.BlockSpec((D, D), lambda i: (0, 0)),
            pl.BlockSpec((D, D), lambda i: (0, 0)),
            pl.BlockSpec((1, D), lambda i: (0, 0)),
        ],
        out_specs=pl.BlockSpec((R, D), lambda i: (i, 0)),
        out_shape=jax.ShapeDtypeStruct((N, D), jnp.float32),
    )(h, aggp, degt, Ws, Wn, b.reshape(1, D))


def kernel(x, edge_index, W_self0, W_neigh0, b0, W_self1, W_neigh1, b1,
           W_self2, W_neigh2, b2):
    src = edge_index[0].reshape(NW, NCHUNK, C)
    dst = edge_index[1].reshape(NW, NCHUNK, C)
    zrows = jnp.zeros((ZR, D), jnp.float32)

    zdeg = jnp.zeros((RPT, 16), jnp.float32)
    ones = jnp.ones((C, 16), jnp.float32)

    degt = _sc_deg(dst, zdeg, ones)
    aggp = _sc_agg(x, src, dst, zrows)
    h = _dense(x, aggp, degt, W_self0, W_neigh0, b0, True)
    aggp = _sc_agg(h, src, dst, zrows)
    h = _dense(h, aggp, degt, W_self1, W_neigh1, b1, True)
    aggp = _sc_agg(h, src, dst, zrows)
    return _dense(h, aggp, degt, W_self2, W_neigh2, b2, False)
